# baseline (speedup 1.0000x reference)
import functools

import jax
import jax.numpy as jnp
from jax import lax
from jax.experimental import pallas as pl
from jax.experimental.pallas import tpu as pltpu
from jax.experimental.pallas import tpu_sc as plsc

_NUM_HEADS = 16
_G = 25
_R0 = 16
_R1 = _G - _R0
_LANES = 16


def _sc_bias_gather(bias_table, rel_idx):
    mesh = plsc.VectorSubcoreMesh(core_axis_name="c", subcore_axis_name="s")

    @functools.partial(
        pl.kernel,
        mesh=mesh,
        out_type=jax.ShapeDtypeStruct((_NUM_HEADS, _G, _G), jnp.float32),
        scratch_types=[
            pltpu.VMEM(bias_table.shape, jnp.float32),
            pltpu.VMEM((_R0, _G), jnp.int32),
            pltpu.VMEM((_R0, _G), jnp.float32),
            pltpu.SemaphoreType.DMA,
            pltpu.SemaphoreType.DMA,
        ],
        compiler_params=pltpu.CompilerParams(needs_layout_passes=False),
    )
    def body(table_hbm, idx_hbm, out_hbm, table_v, idx_v, out_v, sem_t, sem_i):
        wid = lax.axis_index("s") * 2 + lax.axis_index("c")
        head = wid // 2
        second = wid % 2
        hvec = jnp.full((_LANES,), head, dtype=jnp.int32)
        cp_t = pltpu.async_copy(table_hbm, table_v, sem_t)

        def strip(r_base, n_rows):
            cp_i = pltpu.async_copy(
                idx_hbm.at[pl.ds(r_base, n_rows)],
                idx_v.at[pl.ds(0, n_rows)],
                sem_i,
            )
            cp_t.wait()
            cp_i.wait()

            def row(r, carry):
                for c in (0, _G - _LANES):
                    iv = idx_v[r, pl.ds(c, _LANES)]
                    out_v[r, pl.ds(c, _LANES)] = plsc.load_gather(
                        table_v, [iv, hvec]
                    )
                return carry

            lax.fori_loop(0, n_rows, row, 0, unroll=False)
            pltpu.sync_copy(
                out_v.at[pl.ds(0, n_rows)],
                out_hbm.at[head, pl.ds(r_base, n_rows)],
            )

        @pl.when(second == 0)
        def _():
            strip(0, _R0)

        @pl.when(second == 1)
        def _():
            strip(_R0, _R1)

    return body(bias_table, rel_idx)


def kernel(bias_table, rel_idx):
    return _sc_bias_gather(bias_table, rel_idx.astype(jnp.int32))


# speedup vs baseline: 1.1777x; 1.1777x over previous
import functools

import jax
import jax.numpy as jnp
from jax import lax
from jax.experimental import pallas as pl
from jax.experimental.pallas import tpu as pltpu
from jax.experimental.pallas import tpu_sc as plsc

_NUM_HEADS = 16
_G = 25
_LANES = 16


def _sc_bias_gather(bias_table, rel_idx):
    mesh = plsc.VectorSubcoreMesh(
        core_axis_name="c", subcore_axis_name="s", num_cores=1
    )

    @functools.partial(
        pl.kernel,
        mesh=mesh,
        out_type=jax.ShapeDtypeStruct((_NUM_HEADS, _G, _G), jnp.float32),
        scratch_types=[
            pltpu.VMEM(bias_table.shape, jnp.float32),
            pltpu.VMEM((_G, _G), jnp.int32),
            pltpu.VMEM((_G, _G), jnp.float32),
            pltpu.SemaphoreType.DMA,
            pltpu.SemaphoreType.DMA,
        ],
        compiler_params=pltpu.CompilerParams(
            needs_layout_passes=False, use_tc_tiling_on_sc=False
        ),
    )
    def body(table_hbm, idx_hbm, out_hbm, table_v, idx_v, out_v, sem_t, sem_i):
        head = lax.axis_index("s")
        hvec = jnp.full((_LANES,), head, dtype=jnp.int32)
        cp_t = pltpu.async_copy(table_hbm, table_v, sem_t)
        cp_i = pltpu.async_copy(idx_hbm, idx_v, sem_i)
        cp_t.wait()
        cp_i.wait()

        def row(r, carry):
            for c in (0, _G - _LANES):
                iv = idx_v[r, pl.ds(c, _LANES)]
                out_v[r, pl.ds(c, _LANES)] = plsc.load_gather(
                    table_v, [iv, hvec]
                )
            return carry

        lax.fori_loop(0, _G, row, 0, unroll=False)
        pltpu.sync_copy(out_v, out_hbm.at[head])

    return body(bias_table, rel_idx)


def kernel(bias_table, rel_idx):
    return _sc_bias_gather(bias_table, rel_idx.astype(jnp.int32))


# speedup vs baseline: 1.1808x; 1.0026x over previous
import functools

import jax
import jax.numpy as jnp
from jax import lax
from jax.experimental import pallas as pl
from jax.experimental.pallas import tpu as pltpu
from jax.experimental.pallas import tpu_sc as plsc

_NUM_HEADS = 16
_G = 25
_LANES = 16


def _sc_bias_gather(bias_table, rel_idx):
    mesh = plsc.VectorSubcoreMesh(
        core_axis_name="c", subcore_axis_name="s", num_cores=1
    )

    @functools.partial(
        pl.kernel,
        mesh=mesh,
        out_type=jax.ShapeDtypeStruct((_NUM_HEADS, _G, _G), jnp.float32),
        scratch_types=[
            pltpu.VMEM(bias_table.shape, jnp.float32),
            pltpu.VMEM((_G, _G), jnp.int32),
            pltpu.VMEM((_G, _G), jnp.float32),
            pltpu.SemaphoreType.DMA,
            pltpu.SemaphoreType.DMA,
        ],
        compiler_params=pltpu.CompilerParams(
            needs_layout_passes=False,
            use_tc_tiling_on_sc=False,
            skip_device_barrier=True,
        ),
    )
    def body(table_hbm, idx_hbm, out_hbm, table_v, idx_v, out_v, sem_t, sem_i):
        head = lax.axis_index("s")
        hvec = jnp.full((_LANES,), head, dtype=jnp.int32)
        cp_t = pltpu.async_copy(table_hbm, table_v, sem_t)
        cp_i = pltpu.async_copy(idx_hbm, idx_v, sem_i)
        cp_t.wait()
        cp_i.wait()

        def row(r, carry):
            for c in (0, _G - _LANES):
                iv = idx_v[r, pl.ds(c, _LANES)]
                out_v[r, pl.ds(c, _LANES)] = plsc.load_gather(
                    table_v, [iv, hvec]
                )
            return carry

        lax.fori_loop(0, _G, row, 0, unroll=False)
        pltpu.sync_copy(out_v, out_hbm.at[head])

    return body(bias_table, rel_idx)


def kernel(bias_table, rel_idx):
    return _sc_bias_gather(bias_table, rel_idx.astype(jnp.int32))


# speedup vs baseline: 1.1954x; 1.0124x over previous
import functools

import jax
import jax.numpy as jnp
from jax import lax
from jax.experimental import pallas as pl
from jax.experimental.pallas import tpu as pltpu
from jax.experimental.pallas import tpu_sc as plsc

_NUM_HEADS = 16
_G = 25
_LANES = 16


def _sc_bias_gather(bias_table, rel_idx):
    mesh = plsc.VectorSubcoreMesh(
        core_axis_name="c", subcore_axis_name="s", num_cores=1
    )

    @functools.partial(
        pl.kernel,
        mesh=mesh,
        out_type=jax.ShapeDtypeStruct((_NUM_HEADS, _G, _G), jnp.float32),
        scratch_types=[
            pltpu.VMEM(bias_table.shape, jnp.float32),
            pltpu.VMEM((_G, _G), jnp.int32),
            pltpu.VMEM((_G, _G), jnp.float32),
            pltpu.SemaphoreType.DMA,
            pltpu.SemaphoreType.DMA,
        ],
        compiler_params=pltpu.CompilerParams(
            needs_layout_passes=False, use_tc_tiling_on_sc=False
        ),
    )
    def body(table_hbm, idx_hbm, out_hbm, table_v, idx_v, out_v, sem_t, sem_i):
        head = lax.axis_index("s")
        hvec = jnp.full((_LANES,), head, dtype=jnp.int32)
        cp_t = pltpu.async_copy(table_hbm, table_v, sem_t)
        cp_i = pltpu.async_copy(idx_hbm, idx_v, sem_i)
        cp_t.wait()
        cp_i.wait()

        def row(r, carry):
            for c in (0, _G - _LANES):
                iv = idx_v[r, pl.ds(c, _LANES)]
                out_v[r, pl.ds(c, _LANES)] = plsc.load_gather(
                    table_v, [iv, hvec]
                )
            return carry

        lax.fori_loop(0, _G, row, 0, unroll=False)
        pltpu.sync_copy(out_v, out_hbm.at[head])

    return body(bias_table, rel_idx)


def kernel(bias_table, rel_idx):
    return _sc_bias_gather(bias_table, rel_idx.astype(jnp.int32))
